# baseline (speedup 1.0000x reference)
import functools

import jax
import jax.numpy as jnp
from jax import lax
from jax.experimental import pallas as pl
from jax.experimental.pallas import tpu as pltpu
from jax.experimental.pallas import tpu_sc as plsc

B = 16384
EMB = 32
PACK = 4
LANES = EMB * PACK
NU = 1000000
NST = 100000
TBLK = 8192
S_U = 253952
S_S = 32768
NC = 2
NS = 16
NW = NC * NS
BPW = B // NW
CH = 128
NCHUNK = BPW // CH
NFEAT = 4


def _repack_body(x0, x1, x2, x3, eye, out):
    z = jnp.concatenate([x0[...], x1[...], x2[...], x3[...]], axis=0)
    zh = z.astype(jnp.bfloat16)
    zl = (z - zh.astype(jnp.float32)).astype(jnp.bfloat16)
    dims = (((0,), (0,)), ((), ()))
    yh = lax.dot_general(zh, eye[...], dims,
                         preferred_element_type=jnp.float32)
    yl = lax.dot_general(zl, eye[...], dims,
                         preferred_element_type=jnp.float32)
    out[...] = yh + yl


def _repack(t32, n_rows, stride):
    nb = stride // TBLK
    nb_max = -(-n_rows // TBLK) - 1

    def spec(k):
        return pl.BlockSpec(
            (EMB, TBLK), lambda i, k=k: (0, jnp.minimum(i + nb * k, nb_max)))

    eye = jnp.eye(LANES, dtype=jnp.bfloat16)
    return pl.pallas_call(
        _repack_body,
        grid=(nb,),
        in_specs=[spec(0), spec(1), spec(2), spec(3),
                  pl.BlockSpec((LANES, LANES), lambda i: (0, 0))],
        out_specs=pl.BlockSpec((TBLK, LANES), lambda i: (i, 0)),
        out_shape=jax.ShapeDtypeStruct((stride, LANES), jnp.float32),
    )(t32, t32, t32, t32, eye)


def _section_and_row(v, stride):
    one = jnp.int32(1)
    zero = jnp.int32(0)
    k = jnp.where(v >= stride, one, zero)
    k += jnp.where(v >= 2 * stride, one, zero)
    k += jnp.where(v >= 3 * stride, one, zero)
    return k, v - k * jnp.int32(stride)


def _gather_body(utab, stab, uid, sid, f0, f1, f2, f3, x_out,
                 uidx, sidx, uk, sk, fb,
                 bu0, bu1, bs0, bs1, bx0, bx1, gsem, wsem):
    bufu = [bu0, bu1]
    bufs = [bs0, bs1]
    bufx = [bx0, bx1]
    wid = lax.axis_index("s") * NC + lax.axis_index("c")
    base = wid * BPW
    pltpu.sync_copy(uid.at[wid], uidx)
    pltpu.sync_copy(sid.at[wid], sidx)
    for t, f in enumerate((f0, f1, f2, f3)):
        pltpu.sync_copy(f.at[wid], fb.at[t])
    for j in range(NCHUNK):
        for t in range(CH // 16):
            s = pl.ds(t * 16, 16)
            ku, qu = _section_and_row(uidx[j, s], S_U)
            uidx[j, s] = qu
            uk[j, s] = ku * EMB
            ks, qs = _section_and_row(sidx[j, s], S_S)
            sidx[j, s] = qs
            sk[j, s] = ks * EMB
    zeros16 = jnp.zeros((16,), jnp.float32)

    def zbody(r, carry):
        for bx in bufx:
            for c in range(2 * EMB, LANES, 16):
                bx[r, pl.ds(c, 16)] = zeros16
        return carry

    lax.fori_loop(0, CH, zbody, 0)

    rows16 = lax.iota(jnp.int32, 16)

    def compact(j, slot):
        bu, bs, bx = bufu[slot], bufs[slot], bufx[slot]
        jv = jnp.full((16,), j, jnp.int32)

        def body(rg, carry):
            rows = rows16 + rg * 16
            cu = plsc.load_gather(uk, [jv, rows])
            cs = plsc.load_gather(sk, [jv, rows])
            for c in range(EMB):
                v = plsc.load_gather(bu, [rows, cu + c])
                plsc.store_scatter(bx, [rows, jnp.full((16,), c, jnp.int32)], v)
                w = plsc.load_gather(bs, [rows, cs + c])
                plsc.store_scatter(
                    bx, [rows, jnp.full((16,), EMB + c, jnp.int32)], w)
            for t in range(NFEAT):
                fv = fb[t, pl.ds(j * CH + rg * 16, 16)]
                plsc.store_scatter(
                    bx, [rows, jnp.full((16,), 2 * EMB + t, jnp.int32)], fv)
            return carry

        lax.fori_loop(0, CH // 16, body, 0)

    gh = {}
    wh = {}
    for j in range(2):
        gh[j] = (pltpu.async_copy(utab.at[uidx.at[j]], bufu[j], gsem),
                 pltpu.async_copy(stab.at[sidx.at[j]], bufs[j], gsem))
    for j in range(NCHUNK):
        slot = j % 2
        for h in gh[j]:
            h.wait()
        if j >= 2:
            wh[j - 2].wait()
        compact(j, slot)
        if j + 2 < NCHUNK:
            gh[j + 2] = (
                pltpu.async_copy(utab.at[uidx.at[j + 2]], bufu[slot], gsem),
                pltpu.async_copy(stab.at[sidx.at[j + 2]], bufs[slot], gsem))
        wh[j] = pltpu.async_copy(
            bufx[slot], x_out.at[pl.ds(base + j * CH, CH)], wsem)
    wh[NCHUNK - 2].wait()
    wh[NCHUNK - 1].wait()


_sc_gather = pl.kernel(
    _gather_body,
    out_type=jax.ShapeDtypeStruct((B, LANES), jnp.float32),
    mesh=plsc.VectorSubcoreMesh(core_axis_name="c", subcore_axis_name="s"),
    scratch_types=[
        pltpu.VMEM((NCHUNK, CH), jnp.int32),
        pltpu.VMEM((NCHUNK, CH), jnp.int32),
        pltpu.VMEM((NCHUNK, CH), jnp.int32),
        pltpu.VMEM((NCHUNK, CH), jnp.int32),
        pltpu.VMEM((NFEAT, BPW), jnp.float32),
        pltpu.VMEM((CH, LANES), jnp.float32),
        pltpu.VMEM((CH, LANES), jnp.float32),
        pltpu.VMEM((CH, LANES), jnp.float32),
        pltpu.VMEM((CH, LANES), jnp.float32),
        pltpu.VMEM((CH, LANES), jnp.float32),
        pltpu.VMEM((CH, LANES), jnp.float32),
        pltpu.SemaphoreType.DMA,
        pltpu.SemaphoreType.DMA,
    ],
    compiler_params=pltpu.CompilerParams(needs_layout_passes=False),
)

BLK = 4096


def _mlp_body(x, w1p, b1, w2, b2, w3t, b3, out):
    h = jnp.dot(x[...], w1p[...], preferred_element_type=jnp.float32)
    h = jnp.maximum(h + b1[...], 0.0)
    h2 = jnp.dot(h, w2[...], preferred_element_type=jnp.float32)
    h2 = jnp.maximum(h2 + b2[...], 0.0)
    o = jnp.sum(h2 * w3t[...], axis=1) + b3[0, 0]
    out[...] = 1.0 / (1.0 + jnp.exp(-o))


@jax.jit
def kernel(user_id, store_id, sentiment, rating, distance, hour_sin,
           user_table, store_table, W1, b1, W2, b2, W3, b3):
    uid = user_id.astype(jnp.int32)
    sid = store_id.astype(jnp.int32)
    st = _repack(store_table.T, NST, S_S)
    user_t, _ = lax.optimization_barrier((user_table, st))
    ut = _repack(user_t.T, NU, S_U)
    shape_w = (NW, NCHUNK, CH)
    x = _sc_gather(ut, st, uid.reshape(shape_w), sid.reshape(shape_w),
                   sentiment.reshape(NW, BPW), rating.reshape(NW, BPW),
                   distance.reshape(NW, BPW), hour_sin.reshape(NW, BPW))

    w1p = jnp.concatenate(
        [W1, jnp.zeros((LANES - (2 * EMB + NFEAT), 64), jnp.float32)], axis=0)

    full = lambda shape: pl.BlockSpec(shape, lambda i: (0, 0))
    out = pl.pallas_call(
        _mlp_body,
        grid=(B // BLK,),
        in_specs=[
            pl.BlockSpec((BLK, LANES), lambda i: (i, 0)),
            full((LANES, 64)),
            full((1, 64)),
            full((64, 32)),
            full((1, 32)),
            full((1, 32)),
            full((1, 1)),
        ],
        out_specs=pl.BlockSpec((BLK,), lambda i: (i,)),
        out_shape=jax.ShapeDtypeStruct((B,), jnp.float32),
    )(x, w1p, b1.reshape(1, 64), W2, b2.reshape(1, 32), W3.reshape(1, 32),
      b3.reshape(1, 1))
    return out


# speedup vs baseline: 1.1533x; 1.1533x over previous
import functools

import jax
import jax.numpy as jnp
from jax import lax
from jax.experimental import pallas as pl
from jax.experimental.pallas import tpu as pltpu
from jax.experimental.pallas import tpu_sc as plsc

B = 16384
EMB = 32
PACK = 4
LANES = EMB * PACK
NU = 1000000
NST = 100000
TBLK = 8192
S_U = 253952
S_S = 32768
NC = 2
NS = 16
NW = NC * NS
BPW = B // NW
CH = 128
NCHUNK = BPW // CH
NSLOT = 4
NCH_TOT = 2 * NCHUNK


def _repack_body(x0, x1, x2, x3, eye, out):
    z = jnp.concatenate([x0[...], x1[...], x2[...], x3[...]], axis=0)
    zh = z.astype(jnp.bfloat16)
    zl = (z - zh.astype(jnp.float32)).astype(jnp.bfloat16)
    dims = (((0,), (0,)), ((), ()))
    yh = lax.dot_general(zh, eye[...], dims,
                         preferred_element_type=jnp.float32)
    yl = lax.dot_general(zl, eye[...], dims,
                         preferred_element_type=jnp.float32)
    out[...] = yh + yl


def _repack(t32, n_rows, stride):
    nb = stride // TBLK
    nb_max = -(-n_rows // TBLK) - 1

    def spec(k):
        return pl.BlockSpec(
            (EMB, TBLK), lambda i, k=k: (0, jnp.minimum(i + nb * k, nb_max)))

    eye = jnp.eye(LANES, dtype=jnp.bfloat16)
    return pl.pallas_call(
        _repack_body,
        grid=(nb,),
        in_specs=[spec(0), spec(1), spec(2), spec(3),
                  pl.BlockSpec((LANES, LANES), lambda i: (0, 0))],
        out_specs=pl.BlockSpec((TBLK, LANES), lambda i: (i, 0)),
        out_shape=jax.ShapeDtypeStruct((stride, LANES), jnp.float32),
    )(t32, t32, t32, t32, eye)


def _section_and_row(v, stride):
    one = jnp.int32(1)
    zero = jnp.int32(0)
    k = jnp.where(v >= stride, one, zero)
    k += jnp.where(v >= 2 * stride, one, zero)
    k += jnp.where(v >= 3 * stride, one, zero)
    return k, v - k * jnp.int32(stride)


def _make_gather(stride):
    def body(tab, idx, out, idx_v, buf0, buf1, buf2, buf3, gsem, wsem):
        bufs = [buf0, buf1, buf2, buf3]
        wid = lax.axis_index("s") * NC + lax.axis_index("c")
        base = wid * BPW
        pltpu.sync_copy(idx.at[wid], idx_v)
        for j in range(NCHUNK):
            for t in range(CH // 16):
                s = pl.ds(t * 16, 16)
                _, q = _section_and_row(idx_v[j, s], stride)
                idx_v[j, s] = q
        gh = [pltpu.async_copy(tab.at[idx_v.at[j]], bufs[j], gsem)
              for j in range(NCHUNK)]
        wh = []
        for j in range(NCHUNK):
            gh[j].wait()
            wh.append(pltpu.async_copy(
                bufs[j], out.at[pl.ds(base + j * CH, CH)], wsem))
        for w in wh:
            w.wait()

    return pl.kernel(
        body,
        out_type=jax.ShapeDtypeStruct((B, LANES), jnp.float32),
        mesh=plsc.VectorSubcoreMesh(core_axis_name="c", subcore_axis_name="s"),
        scratch_types=[
            pltpu.VMEM((NCHUNK, CH), jnp.int32),
            pltpu.VMEM((CH, LANES), jnp.float32),
            pltpu.VMEM((CH, LANES), jnp.float32),
            pltpu.VMEM((CH, LANES), jnp.float32),
            pltpu.VMEM((CH, LANES), jnp.float32),
            pltpu.SemaphoreType.DMA,
            pltpu.SemaphoreType.DMA,
        ],
    )


_gather_user = _make_gather(S_U)
_gather_store = _make_gather(S_S)

BLK = 4096


def _dot3(a, b):
    ah = a.astype(jnp.bfloat16)
    al = (a - ah.astype(jnp.float32)).astype(jnp.bfloat16)
    bh = b.astype(jnp.bfloat16)
    bl = (b - bh.astype(jnp.float32)).astype(jnp.bfloat16)
    pet = jnp.float32
    return (jnp.dot(ah, bh, preferred_element_type=pet) +
            (jnp.dot(ah, bl, preferred_element_type=pet) +
             jnp.dot(al, bh, preferred_element_type=pet)))


def _mlp_body(uid, sid, ug, sg, f, w1u4, w1s4, w1f, b1, w2, b2, w3t, b3, out):
    lane = lax.broadcasted_iota(jnp.int32, (BLK, LANES), 1) // EMB
    ku, _ = _section_and_row(uid[...], S_U)
    ks, _ = _section_and_row(sid[...], S_S)
    um = (lane == ku).astype(jnp.float32)
    sm = (lane == ks).astype(jnp.float32)
    h = jnp.dot(ug[...] * um, w1u4[...], preferred_element_type=jnp.float32)
    h += jnp.dot(sg[...] * sm, w1s4[...], preferred_element_type=jnp.float32)
    h += lax.dot_general(f[...], w1f[...], (((0,), (0,)), ((), ())),
                         preferred_element_type=jnp.float32)
    h = jnp.maximum(h + b1[...], 0.0)
    h2 = jnp.dot(h, w2[...], preferred_element_type=jnp.float32)
    h2 = jnp.maximum(h2 + b2[...], 0.0)
    o = jnp.sum(h2 * w3t[...], axis=1) + b3[0, 0]
    out[...] = 1.0 / (1.0 + jnp.exp(-o))


def _rows(i):
    return (i, 0)


@jax.jit
def kernel(user_id, store_id, sentiment, rating, distance, hour_sin,
           user_table, store_table, W1, b1, W2, b2, W3, b3):
    uid = user_id.astype(jnp.int32)
    sid = store_id.astype(jnp.int32)
    st = _repack(store_table.T, NST, S_S)
    sg = _gather_store(st, sid.reshape(NW, NCHUNK, CH))
    user_t, _ = lax.optimization_barrier((user_table, st))
    ut = _repack(user_t.T, NU, S_U)
    ug = _gather_user(ut, uid.reshape(NW, NCHUNK, CH))

    f = jnp.stack([sentiment, rating, distance, hour_sin], axis=0)
    w1u4 = jnp.concatenate([W1[:EMB]] * PACK, axis=0)
    w1s4 = jnp.concatenate([W1[EMB:2 * EMB]] * PACK, axis=0)
    w1f = W1[2 * EMB:]

    full = lambda shape: pl.BlockSpec(shape, lambda i: (0, 0))
    out = pl.pallas_call(
        _mlp_body,
        grid=(B // BLK,),
        in_specs=[
            pl.BlockSpec((BLK, 1), _rows),
            pl.BlockSpec((BLK, 1), _rows),
            pl.BlockSpec((BLK, LANES), _rows),
            pl.BlockSpec((BLK, LANES), _rows),
            pl.BlockSpec((4, BLK), lambda i: (0, i)),
            full((LANES, 64)),
            full((LANES, 64)),
            full((4, 64)),
            full((1, 64)),
            full((64, 32)),
            full((1, 32)),
            full((1, 32)),
            full((1, 1)),
        ],
        out_specs=pl.BlockSpec((BLK,), lambda i: (i,)),
        out_shape=jax.ShapeDtypeStruct((B,), jnp.float32),
    )(uid.reshape(B, 1), sid.reshape(B, 1), ug, sg, f, w1u4, w1s4, w1f,
      b1.reshape(1, 64), W2, b2.reshape(1, 32), W3.reshape(1, 32),
      b3.reshape(1, 1))
    return out
